# Initial kernel scaffold; baseline (speedup 1.0000x reference)
import jax
import jax.numpy as jnp
from jax.experimental import pallas as pl


def kernel(x, edge_index, batch, W1, b1, W2, b2, Wl, bl):
    raise NotImplementedError("write your pallas kernel here")


# speedup vs baseline: 11.3271x; 11.3271x over previous
import functools

import jax
import jax.numpy as jnp
from jax import lax
from jax.experimental import pallas as pl
from jax.experimental.pallas import tpu as pltpu
from jax.experimental.pallas import tpu_sc as plsc

N = 10000
E = 320000
F_IN = 128
G = 128
F1 = 16
NC = 2
NS = 16
NW = NC * NS
EW = E // NW
CH = 80
NCH = EW // CH
NPAD = 10240
SLC = NPAD // NS

_MESH = plsc.VectorSubcoreMesh(core_axis_name="c", subcore_axis_name="s",
                               num_cores=NC, num_subcores=NS)
_SC_PARAMS = pltpu.CompilerParams(use_tc_tiling_on_sc=False)


def _deg_body(dst_hbm, ones_hbm, zeros_hbm, out_hbm, idx_v, ones_v, deg_sh):
    c = lax.axis_index("c")
    s = lax.axis_index("s")
    base = (s * NC + c) * EW
    pltpu.sync_copy(ones_hbm, ones_v)
    pltpu.sync_copy(zeros_hbm, deg_sh.at[pl.ds(s * SLC, SLC)])
    plsc.subcore_barrier()

    def body(j, carry):
        pltpu.sync_copy(dst_hbm.at[pl.ds(base + j * CH, CH)], idx_v)
        pltpu.sync_copy(ones_v, deg_sh.at[idx_v], add=True)
        return carry

    lax.fori_loop(0, NCH, body, 0)
    plsc.subcore_barrier()
    pltpu.sync_copy(deg_sh.at[pl.ds(s * SLC, SLC)],
                    out_hbm.at[c, pl.ds(s * SLC, SLC)])


_deg_call = pl.kernel(
    _deg_body,
    out_type=jax.ShapeDtypeStruct((NC, NPAD), jnp.float32),
    mesh=_MESH,
    scratch_types=[
        pltpu.VMEM((CH,), jnp.int32),
        pltpu.VMEM((CH,), jnp.float32),
        pltpu.VMEM_SHARED((NPAD,), jnp.float32),
    ],
    compiler_params=_SC_PARAMS,
)


def _edge_body(q_hbm, src_hbm, dst_hbm, zeros_hbm, out_hbm,
               idxs_v, idxd_v, rows_v, acc_sh, sem):
    c = lax.axis_index("c")
    s = lax.axis_index("s")
    base = (s * NC + c) * EW
    pltpu.sync_copy(zeros_hbm, acc_sh.at[pl.ds(s * SLC, SLC)])
    plsc.subcore_barrier()

    def body(j, carry):
        off = base + j * CH
        pltpu.sync_copy(src_hbm.at[pl.ds(off, CH)], idxs_v)
        pltpu.sync_copy(dst_hbm.at[pl.ds(off, CH)], idxd_v)
        pltpu.async_copy(q_hbm.at[idxs_v], rows_v, sem).wait()
        pltpu.sync_copy(rows_v, acc_sh.at[idxd_v], add=True)
        return carry

    lax.fori_loop(0, NCH, body, 0)
    plsc.subcore_barrier()
    pltpu.sync_copy(acc_sh.at[pl.ds(s * SLC, SLC)],
                    out_hbm.at[c, pl.ds(s * SLC, SLC)])


_edge_call = pl.kernel(
    _edge_body,
    out_type=jax.ShapeDtypeStruct((NC, NPAD, F1), jnp.float32),
    mesh=_MESH,
    scratch_types=[
        pltpu.VMEM((CH,), jnp.int32),
        pltpu.VMEM((CH,), jnp.int32),
        pltpu.VMEM((CH, F1), jnp.float32),
        pltpu.VMEM_SHARED((NPAD, F1), jnp.float32),
        pltpu.SemaphoreType.DMA,
    ],
    compiler_params=_SC_PARAMS,
)


def _k2_body(x_ref, dp_ref, w1_ref, q1_ref, dinv_ref):
    deg = dp_ref[:, 0:1] + dp_ref[:, 1:2] + 1.0
    dinv = lax.rsqrt(deg)
    oh = (x_ref[...] == lax.broadcasted_iota(jnp.int32, (N, F_IN), 1))
    q1 = jnp.dot(oh.astype(jnp.float32), w1_ref[...],
                 preferred_element_type=jnp.float32)
    q1_ref[...] = q1 * dinv
    dinv_ref[...] = dinv


_k2_call = pl.pallas_call(
    _k2_body,
    out_shape=(jax.ShapeDtypeStruct((N, F1), jnp.float32),
               jax.ShapeDtypeStruct((N, 1), jnp.float32)),
)


def _k4_body(q1_ref, a0_ref, a1_ref, dinv_ref, b1_ref, w2_ref, q2_ref):
    dinv = dinv_ref[...]
    s1 = q1_ref[...] + a0_ref[...] + a1_ref[...]
    h1 = jnp.maximum(s1 * dinv + b1_ref[...], 0.0)
    q2_ref[...] = jnp.dot(h1, w2_ref[...],
                          preferred_element_type=jnp.float32) * dinv


_k4_call = pl.pallas_call(
    _k4_body,
    out_shape=jax.ShapeDtypeStruct((N, F1), jnp.float32),
)


def _k6a_body(q2_ref, a0_ref, a1_ref, dinv_ref, b2_ref, h2_ref):
    h2_ref[...] = ((q2_ref[...] + a0_ref[...] + a1_ref[...]) * dinv_ref[...]
                   + b2_ref[...])


_k6a_call = pl.pallas_call(
    _k6a_body,
    out_shape=jax.ShapeDtypeStruct((N, F1), jnp.float32),
)


def _k6b_body(h2_ref, batch_ref, out_ref):
    j = pl.program_id(0)
    mask = batch_ref[...] == j
    m = jnp.max(jnp.where(mask, h2_ref[...], -3.0e38), axis=0)
    out_ref[...] = m.reshape(1, 1, F1)


_k6b_call = pl.pallas_call(
    _k6b_body,
    grid=(G,),
    in_specs=[
        pl.BlockSpec((N, F1), lambda j: (0, 0)),
        pl.BlockSpec((N, 1), lambda j: (0, 0)),
    ],
    out_specs=pl.BlockSpec((1, 1, F1), lambda j: (j, 0, 0)),
    out_shape=jax.ShapeDtypeStruct((G, 1, F1), jnp.float32),
)


def _k7_body(g_ref, wlt_ref, bl_ref, out_ref):
    out_ref[...] = jnp.dot(g_ref[...], wlt_ref[...],
                           preferred_element_type=jnp.float32) + bl_ref[...]


_k7_call = pl.pallas_call(
    _k7_body,
    out_shape=jax.ShapeDtypeStruct((G, 1), jnp.float32),
)


_TRUNC = 0


def kernel(x, edge_index, batch, W1, b1, W2, b2, Wl, bl):
    x = x.astype(jnp.int32)
    ei = edge_index.astype(jnp.int32)
    batch = batch.astype(jnp.int32)
    src, dst = ei[0], ei[1]

    ones_c = jnp.ones((CH,), jnp.float32)
    zeros1 = jnp.zeros((SLC,), jnp.float32)
    zeros2 = jnp.zeros((SLC, F1), jnp.float32)

    degp = _deg_call(dst, ones_c, zeros1)
    if _TRUNC == 1:
        return degp.reshape(-1)[:G]
    dp = jnp.stack([degp[0, :N], degp[1, :N]], axis=1)
    q1, dinv = _k2_call(x.reshape(N, 1), dp, W1)
    if _TRUNC == 2:
        return q1.reshape(-1)[:G]

    acc1 = _edge_call(q1, src, dst, zeros2)
    if _TRUNC == 3:
        return acc1.reshape(-1)[:G]
    W2p = jnp.zeros((F1, F1), jnp.float32).at[:, :5].set(W2)
    q2 = _k4_call(q1, acc1[0, :N], acc1[1, :N], dinv,
                  b1.reshape(1, F1), W2p)
    if _TRUNC == 4:
        return q2.reshape(-1)[:G]

    acc2 = _edge_call(q2, src, dst, zeros2)
    b2p = jnp.zeros((1, F1), jnp.float32).at[0, :5].set(b2)
    wlt = jnp.zeros((F1, 1), jnp.float32).at[:5, 0].set(Wl[0])
    h2 = _k6a_call(q2, acc2[0, :N], acc2[1, :N], dinv, b2p)
    gmax = _k6b_call(h2, batch.reshape(N, 1))
    out = _k7_call(gmax.reshape(G, F1), wlt, bl.reshape(1, 1))
    return out.reshape(-1)


# speedup vs baseline: 23.6412x; 2.0871x over previous
import functools

import jax
import jax.numpy as jnp
from jax import lax
from jax.experimental import pallas as pl
from jax.experimental.pallas import tpu as pltpu
from jax.experimental.pallas import tpu_sc as plsc

N = 10000
E = 320000
F_IN = 128
G = 128
F1 = 16
NC = 2
NS = 16
NW = NC * NS
EW = E // NW
CH = 2000
NCH = EW // CH
NPAD = 10240
SLC = NPAD // NS

_MESH = plsc.VectorSubcoreMesh(core_axis_name="c", subcore_axis_name="s",
                               num_cores=NC, num_subcores=NS)
_SC_PARAMS = pltpu.CompilerParams(use_tc_tiling_on_sc=False)


def _deg_body(dst_hbm, ones_hbm, zeros_hbm, out_hbm, idx_v, ones_v, deg_sh):
    c = lax.axis_index("c")
    s = lax.axis_index("s")
    base = (s * NC + c) * EW
    pltpu.sync_copy(ones_hbm, ones_v)
    pltpu.sync_copy(zeros_hbm, deg_sh.at[pl.ds(s * SLC, SLC)])
    plsc.subcore_barrier()

    def body(j, carry):
        pltpu.sync_copy(dst_hbm.at[pl.ds(base + j * CH, CH)], idx_v)
        pltpu.sync_copy(ones_v, deg_sh.at[idx_v], add=True)
        return carry

    lax.fori_loop(0, NCH, body, 0)
    plsc.subcore_barrier()
    pltpu.sync_copy(deg_sh.at[pl.ds(s * SLC, SLC)],
                    out_hbm.at[c, pl.ds(s * SLC, SLC)])


_deg_call = pl.kernel(
    _deg_body,
    out_type=jax.ShapeDtypeStruct((NC, NPAD), jnp.float32),
    mesh=_MESH,
    scratch_types=[
        pltpu.VMEM((CH,), jnp.int32),
        pltpu.VMEM((CH,), jnp.float32),
        pltpu.VMEM_SHARED((NPAD,), jnp.float32),
    ],
    compiler_params=_SC_PARAMS,
)


def _edge_body(q_hbm, src_hbm, dst_hbm, zeros_hbm, out_hbm,
               idxs_v, idxd_v, rows_v, acc_sh, sem):
    c = lax.axis_index("c")
    s = lax.axis_index("s")
    base = (s * NC + c) * EW
    pltpu.sync_copy(zeros_hbm, acc_sh.at[pl.ds(s * SLC, SLC)])
    plsc.subcore_barrier()

    def body(j, carry):
        off = base + j * CH
        pltpu.sync_copy(src_hbm.at[pl.ds(off, CH)], idxs_v)
        pltpu.sync_copy(dst_hbm.at[pl.ds(off, CH)], idxd_v)
        pltpu.async_copy(q_hbm.at[idxs_v], rows_v, sem).wait()
        pltpu.sync_copy(rows_v, acc_sh.at[idxd_v], add=True)
        return carry

    lax.fori_loop(0, NCH, body, 0)
    plsc.subcore_barrier()
    pltpu.sync_copy(acc_sh.at[pl.ds(s * SLC, SLC)],
                    out_hbm.at[c, pl.ds(s * SLC, SLC)])


_edge_call = pl.kernel(
    _edge_body,
    out_type=jax.ShapeDtypeStruct((NC, NPAD, F1), jnp.float32),
    mesh=_MESH,
    scratch_types=[
        pltpu.VMEM((CH,), jnp.int32),
        pltpu.VMEM((CH,), jnp.int32),
        pltpu.VMEM((CH, F1), jnp.float32),
        pltpu.VMEM_SHARED((NPAD, F1), jnp.float32),
        pltpu.SemaphoreType.DMA,
    ],
    compiler_params=_SC_PARAMS,
)


def _k2_body(x_ref, dp_ref, w1_ref, q1_ref, dinv_ref):
    deg = dp_ref[:, 0:1] + dp_ref[:, 1:2] + 1.0
    dinv = lax.rsqrt(deg)
    oh = (x_ref[...] == lax.broadcasted_iota(jnp.int32, (N, F_IN), 1))
    q1 = jnp.dot(oh.astype(jnp.float32), w1_ref[...],
                 preferred_element_type=jnp.float32)
    q1_ref[...] = q1 * dinv
    dinv_ref[...] = dinv


_k2_call = pl.pallas_call(
    _k2_body,
    out_shape=(jax.ShapeDtypeStruct((N, F1), jnp.float32),
               jax.ShapeDtypeStruct((N, 1), jnp.float32)),
)


def _k4_body(q1_ref, a0_ref, a1_ref, dinv_ref, b1_ref, w2_ref, q2_ref):
    dinv = dinv_ref[...]
    s1 = q1_ref[...] + a0_ref[...] + a1_ref[...]
    h1 = jnp.maximum(s1 * dinv + b1_ref[...], 0.0)
    q2_ref[...] = jnp.dot(h1, w2_ref[...],
                          preferred_element_type=jnp.float32) * dinv


_k4_call = pl.pallas_call(
    _k4_body,
    out_shape=jax.ShapeDtypeStruct((N, F1), jnp.float32),
)


def _k6a_body(q2_ref, a0_ref, a1_ref, dinv_ref, b2_ref, h2_ref):
    h2_ref[...] = ((q2_ref[...] + a0_ref[...] + a1_ref[...]) * dinv_ref[...]
                   + b2_ref[...])


_k6a_call = pl.pallas_call(
    _k6a_body,
    out_shape=jax.ShapeDtypeStruct((N, F1), jnp.float32),
)


def _k6b_body(h2_ref, batch_ref, out_ref):
    j = pl.program_id(0)
    mask = batch_ref[...] == j
    m = jnp.max(jnp.where(mask, h2_ref[...], -3.0e38), axis=0)
    out_ref[...] = m.reshape(1, 1, F1)


_k6b_call = pl.pallas_call(
    _k6b_body,
    grid=(G,),
    in_specs=[
        pl.BlockSpec((N, F1), lambda j: (0, 0)),
        pl.BlockSpec((N, 1), lambda j: (0, 0)),
    ],
    out_specs=pl.BlockSpec((1, 1, F1), lambda j: (j, 0, 0)),
    out_shape=jax.ShapeDtypeStruct((G, 1, F1), jnp.float32),
)


def _k7_body(g_ref, wlt_ref, bl_ref, out_ref):
    out_ref[...] = jnp.dot(g_ref[...], wlt_ref[...],
                           preferred_element_type=jnp.float32) + bl_ref[...]


_k7_call = pl.pallas_call(
    _k7_body,
    out_shape=jax.ShapeDtypeStruct((G, 1), jnp.float32),
)


_TRUNC = 0


def kernel(x, edge_index, batch, W1, b1, W2, b2, Wl, bl):
    x = x.astype(jnp.int32)
    ei = edge_index.astype(jnp.int32)
    batch = batch.astype(jnp.int32)
    src, dst = ei[0], ei[1]

    ones_c = jnp.ones((CH,), jnp.float32)
    zeros1 = jnp.zeros((SLC,), jnp.float32)
    zeros2 = jnp.zeros((SLC, F1), jnp.float32)

    degp = _deg_call(dst, ones_c, zeros1)
    if _TRUNC == 1:
        return degp.reshape(-1)[:G]
    dp = jnp.stack([degp[0, :N], degp[1, :N]], axis=1)
    q1, dinv = _k2_call(x.reshape(N, 1), dp, W1)
    if _TRUNC == 2:
        return q1.reshape(-1)[:G]

    acc1 = _edge_call(q1, src, dst, zeros2)
    if _TRUNC == 3:
        return acc1.reshape(-1)[:G]
    W2p = jnp.zeros((F1, F1), jnp.float32).at[:, :5].set(W2)
    q2 = _k4_call(q1, acc1[0, :N], acc1[1, :N], dinv,
                  b1.reshape(1, F1), W2p)
    if _TRUNC == 4:
        return q2.reshape(-1)[:G]

    acc2 = _edge_call(q2, src, dst, zeros2)
    b2p = jnp.zeros((1, F1), jnp.float32).at[0, :5].set(b2)
    wlt = jnp.zeros((F1, 1), jnp.float32).at[:5, 0].set(Wl[0])
    h2 = _k6a_call(q2, acc2[0, :N], acc2[1, :N], dinv, b2p)
    gmax = _k6b_call(h2, batch.reshape(N, 1))
    out = _k7_call(gmax.reshape(G, F1), wlt, bl.reshape(1, 1))
    return out.reshape(-1)


# speedup vs baseline: 25.6039x; 1.0830x over previous
import functools

import jax
import jax.numpy as jnp
from jax import lax
from jax.experimental import pallas as pl
from jax.experimental.pallas import tpu as pltpu
from jax.experimental.pallas import tpu_sc as plsc

N = 10000
E = 320000
F_IN = 128
G = 128
F1 = 16
F2 = 8
NC = 2
NS = 16
NW = NC * NS
EW = E // NW
NPAD = 10240
SLC = NPAD // NS

_MESH = plsc.VectorSubcoreMesh(core_axis_name="c", subcore_axis_name="s",
                               num_cores=NC, num_subcores=NS)
_SC_PARAMS = pltpu.CompilerParams(use_tc_tiling_on_sc=False)


def _deg_body(dst_hbm, ones_hbm, zeros_hbm, out_hbm, idx_v, ones_v, deg_sh):
    c = lax.axis_index("c")
    s = lax.axis_index("s")
    base = (s * NC + c) * EW
    pltpu.sync_copy(ones_hbm, ones_v)
    pltpu.sync_copy(zeros_hbm, deg_sh.at[pl.ds(s * SLC, SLC)])
    plsc.subcore_barrier()
    pltpu.sync_copy(dst_hbm.at[pl.ds(base, EW)], idx_v)
    pltpu.sync_copy(ones_v, deg_sh.at[idx_v], add=True)
    plsc.subcore_barrier()
    pltpu.sync_copy(deg_sh.at[pl.ds(s * SLC, SLC)],
                    out_hbm.at[c, pl.ds(s * SLC, SLC)])


_deg_call = pl.kernel(
    _deg_body,
    out_type=jax.ShapeDtypeStruct((NC, NPAD), jnp.float32),
    mesh=_MESH,
    scratch_types=[
        pltpu.VMEM((EW,), jnp.int32),
        pltpu.VMEM((EW,), jnp.float32),
        pltpu.VMEM_SHARED((NPAD,), jnp.float32),
    ],
    compiler_params=_SC_PARAMS,
)


def _make_edge_call(f, ch):
    nch = EW // ch

    def _edge_body(q_hbm, src_hbm, dst_hbm, zeros_hbm, out_hbm,
                   idxs_v, idxd_v, rows_v, acc_sh, sem):
        c = lax.axis_index("c")
        s = lax.axis_index("s")
        base = (s * NC + c) * EW
        pltpu.sync_copy(zeros_hbm, acc_sh.at[pl.ds(s * SLC, SLC)])
        plsc.subcore_barrier()

        def body(j, carry):
            off = base + j * ch
            pltpu.sync_copy(src_hbm.at[pl.ds(off, ch)], idxs_v)
            pltpu.sync_copy(dst_hbm.at[pl.ds(off, ch)], idxd_v)
            pltpu.async_copy(q_hbm.at[idxs_v], rows_v, sem).wait()
            pltpu.sync_copy(rows_v, acc_sh.at[idxd_v], add=True)
            return carry

        lax.fori_loop(0, nch, body, 0)
        plsc.subcore_barrier()
        pltpu.sync_copy(acc_sh.at[pl.ds(s * SLC, SLC)],
                        out_hbm.at[c, pl.ds(s * SLC, SLC)])

    return pl.kernel(
        _edge_body,
        out_type=jax.ShapeDtypeStruct((NC, NPAD, f), jnp.float32),
        mesh=_MESH,
        scratch_types=[
            pltpu.VMEM((ch,), jnp.int32),
            pltpu.VMEM((ch,), jnp.int32),
            pltpu.VMEM((ch, f), jnp.float32),
            pltpu.VMEM_SHARED((NPAD, f), jnp.float32),
            pltpu.SemaphoreType.DMA,
        ],
        compiler_params=_SC_PARAMS,
    )


_edge_call_1 = _make_edge_call(F1, 5000)
_edge_call_2 = _make_edge_call(F2, 10000)


def _k2_body(x_ref, dp_ref, w1_ref, q1_ref, dinv_ref):
    deg = dp_ref[:, 0:1] + dp_ref[:, 1:2] + 1.0
    dinv = lax.rsqrt(deg)
    oh = (x_ref[...] == lax.broadcasted_iota(jnp.int32, (N, F_IN), 1))
    q1 = jnp.dot(oh.astype(jnp.float32), w1_ref[...],
                 preferred_element_type=jnp.float32)
    q1_ref[...] = q1 * dinv
    dinv_ref[...] = dinv


_k2_call = pl.pallas_call(
    _k2_body,
    out_shape=(jax.ShapeDtypeStruct((N, F1), jnp.float32),
               jax.ShapeDtypeStruct((N, 1), jnp.float32)),
)


def _k4_body(q1_ref, acc_ref, dinv_ref, b1_ref, w2_ref, q2_ref):
    dinv = dinv_ref[...]
    s1 = q1_ref[...] + acc_ref[0, :N, :] + acc_ref[1, :N, :]
    h1 = jnp.maximum(s1 * dinv + b1_ref[...], 0.0)
    q2_ref[...] = jnp.dot(h1, w2_ref[...],
                          preferred_element_type=jnp.float32) * dinv


_k4_call = pl.pallas_call(
    _k4_body,
    out_shape=jax.ShapeDtypeStruct((N, F2), jnp.float32),
)


def _k6a_body(q2_ref, acc_ref, dinv_ref, b2_ref, h2_ref):
    h2_ref[...] = ((q2_ref[...] + acc_ref[0, :N, :] + acc_ref[1, :N, :])
                   * dinv_ref[...] + b2_ref[...])


_k6a_call = pl.pallas_call(
    _k6a_body,
    out_shape=jax.ShapeDtypeStruct((N, F2), jnp.float32),
)


def _k6b_body(h2_ref, batch_ref, out_ref):
    j = pl.program_id(0)
    mask = batch_ref[...] == j
    m = jnp.max(jnp.where(mask, h2_ref[...], -3.0e38), axis=0)
    out_ref[...] = m.reshape(1, 1, F2)


_k6b_call = pl.pallas_call(
    _k6b_body,
    grid=(G,),
    in_specs=[
        pl.BlockSpec((N, F2), lambda j: (0, 0)),
        pl.BlockSpec((N, 1), lambda j: (0, 0)),
    ],
    out_specs=pl.BlockSpec((1, 1, F2), lambda j: (j, 0, 0)),
    out_shape=jax.ShapeDtypeStruct((G, 1, F2), jnp.float32),
)


def _k7_body(g_ref, wlt_ref, bl_ref, out_ref):
    out_ref[...] = jnp.dot(g_ref[...], wlt_ref[...],
                           preferred_element_type=jnp.float32) + bl_ref[...]


_k7_call = pl.pallas_call(
    _k7_body,
    out_shape=jax.ShapeDtypeStruct((G, 1), jnp.float32),
)


def kernel(x, edge_index, batch, W1, b1, W2, b2, Wl, bl):
    x = x.astype(jnp.int32)
    ei = edge_index.astype(jnp.int32)
    batch = batch.astype(jnp.int32)
    src, dst = ei[0], ei[1]

    ones_c = jnp.ones((EW,), jnp.float32)
    zeros1 = jnp.zeros((SLC,), jnp.float32)
    zeros_f1 = jnp.zeros((SLC, F1), jnp.float32)
    zeros_f2 = jnp.zeros((SLC, F2), jnp.float32)

    degp = _deg_call(dst, ones_c, zeros1)
    dp = jnp.stack([degp[0, :N], degp[1, :N]], axis=1)
    q1, dinv = _k2_call(x.reshape(N, 1), dp, W1)

    acc1 = _edge_call_1(q1, src, dst, zeros_f1)
    W2p = jnp.zeros((F1, F2), jnp.float32).at[:, :5].set(W2)
    q2 = _k4_call(q1, acc1, dinv, b1.reshape(1, F1), W2p)

    acc2 = _edge_call_2(q2, src, dst, zeros_f2)
    b2p = jnp.zeros((1, F2), jnp.float32).at[0, :5].set(b2)
    wlt = jnp.zeros((F2, 1), jnp.float32).at[:5, 0].set(Wl[0])
    h2 = _k6a_call(q2, acc2, dinv, b2p)
    gmax = _k6b_call(h2, batch.reshape(N, 1))
    out = _k7_call(gmax.reshape(G, F2), wlt, bl.reshape(1, 1))
    return out.reshape(-1)
